# baseline (speedup 1.0000x reference)
import functools
import jax
import jax.numpy as jnp
from jax import lax
from jax.experimental import pallas as pl
from jax.experimental.pallas import tpu as pltpu
from jax.experimental.pallas import tpu_sc as plsc

N = 10000
E = 320000
D = 128
G = 16
NC = 2
NS = 16
NW = NC * NS
CH = 64
NCH = 160
DCH = 128
DNCH = 80
EP = NW * NCH * CH
NP = 10240
NPT = NP // NS
RB = 32

_mesh = plsc.VectorSubcoreMesh(core_axis_name="c", subcore_axis_name="s")

_GDN = lax.GatherDimensionNumbers(
    offset_dims=(), collapsed_slice_dims=(0,), start_index_map=(0,))


def _lane_bcast(vec16, r):
    idx = jnp.full((16, 1), r, jnp.int32)
    return lax.gather(vec16, idx, _GDN, (1,),
                      mode=lax.GatherScatterMode.PROMISE_IN_BOUNDS)


@functools.partial(
    pl.kernel,
    out_type=jax.ShapeDtypeStruct((NC, NP), jnp.float32),
    mesh=_mesh,
    scratch_types=[
        pltpu.VMEM((DNCH, DCH), jnp.int32),
        pltpu.VMEM((DNCH, DCH), jnp.float32),
        pltpu.VMEM((NPT,), jnp.float32),
        pltpu.VMEM_SHARED((NP,), jnp.float32),
    ],
)
def _sc_degree(dst_hbm, ew_hbm, out_hbm, dst_v, ew_v, buf_v, deg_sh):
    c = lax.axis_index("c")
    s = lax.axis_index("s")
    wid = c * NS + s

    for k in range(NPT // 16):
        buf_v[pl.ds(k * 16, 16)] = jnp.zeros((16,), jnp.float32)
    pltpu.sync_copy(buf_v, deg_sh.at[pl.ds(s * NPT, NPT)])
    plsc.subcore_barrier()

    row0 = wid * DNCH
    pltpu.sync_copy(dst_hbm.at[pl.ds(row0, DNCH), :], dst_v)
    pltpu.sync_copy(ew_hbm.at[pl.ds(row0, DNCH), :], ew_v)

    def body(j, carry):
        pltpu.sync_copy(ew_v.at[j], deg_sh.at[dst_v.at[j]], add=True)
        return carry

    lax.fori_loop(0, DNCH, body, 0)
    plsc.subcore_barrier()

    pltpu.sync_copy(deg_sh.at[pl.ds(s * NPT, NPT)], buf_v)
    pltpu.sync_copy(buf_v, out_hbm.at[c, pl.ds(s * NPT, NPT)])


NBUF = 4
NSB = 2
N0 = 204
N1 = 116
NMX = max(N0, N1)

_msg_scratch = (
    [pltpu.VMEM((NMX * CH,), jnp.float32)]
    + [pltpu.VMEM((CH,), jnp.int32) for _ in range(NBUF)]
    + [pltpu.VMEM((CH,), jnp.int32) for _ in range(NBUF)]
    + [pltpu.VMEM((CH, D // 2), jnp.int32) for _ in range(NBUF)]
    + [pltpu.VMEM((CH, D), jnp.float32) for _ in range(NSB)]
    + [pltpu.VMEM_SHARED((NP, D), jnp.float32)]
    + [pltpu.SemaphoreType.DMA for _ in range(3 * NBUF + NSB)]
)


@functools.partial(
    pl.kernel,
    out_type=jax.ShapeDtypeStruct((NC, NP, D), jnp.float32),
    mesh=_mesh,
    scratch_types=_msg_scratch,
    compiler_params=pltpu.CompilerParams(needs_layout_passes=False, use_tc_tiling_on_sc=False),
)
def _sc_messages(src_hbm, dst_hbm, ew_hbm, hpb_hbm, out_hbm, ew_v, *bufs):
    srcs = bufs[0:NBUF]
    dsts = bufs[NBUF:2 * NBUF]
    bfr = bufs[2 * NBUF:3 * NBUF]
    f32r = bufs[3 * NBUF:3 * NBUF + NSB]
    acc_sh = bufs[3 * NBUF + NSB]
    o = 3 * NBUF + NSB + 1
    semg = bufs[o:o + NBUF]
    semis = bufs[o + NBUF:o + 2 * NBUF]
    semid = bufs[o + 2 * NBUF:o + 3 * NBUF]
    sems = bufs[o + 3 * NBUF:o + 3 * NBUF + NSB]

    c = lax.axis_index("c")
    s = lax.axis_index("s")
    nch = jnp.where(c == 0, N0, N1)
    row0 = jnp.where(c == 0, s * N0, NS * N0 + s * N1)

    for r in range(RB):
        for k in range(D // 16):
            f32r[0][r, pl.ds(k * 16, 16)] = jnp.zeros((16,), jnp.float32)
    for m in range(NPT // RB):
        pltpu.sync_copy(f32r[0].at[pl.ds(0, RB), :],
                        acc_sh.at[pl.ds(s * NPT + m * RB, RB), :])
    plsc.subcore_barrier()

    pltpu.sync_copy(ew_hbm.at[pl.ds(row0 * CH, NMX * CH)], ew_v)

    def _isrc(j, p):
        pltpu.async_copy(src_hbm.at[pl.ds((row0 + j) * CH, CH)],
                         srcs[p], semis[p])

    def _idst(j, p):
        pltpu.async_copy(dst_hbm.at[pl.ds((row0 + j) * CH, CH)],
                         dsts[p], semid[p])

    def _gather(p):
        pltpu.async_copy(hpb_hbm.at[srcs[p]], bfr[p], semg[p])

    def _scatter(p, p2):
        pltpu.async_copy(f32r[p2], acc_sh.at[dsts[p]], sems[p2], add=True)

    def _wait_isrc(p):
        pltpu.make_async_copy(src_hbm.at[pl.ds(0, CH)], srcs[p],
                              semis[p]).wait()

    def _wait_idst(p):
        pltpu.make_async_copy(dst_hbm.at[pl.ds(0, CH)], dsts[p],
                              semid[p]).wait()

    def _wait_gather(p):
        pltpu.make_async_copy(hpb_hbm.at[srcs[p]], bfr[p], semg[p]).wait()

    def _wait_scatter(p2):
        pltpu.make_async_copy(f32r[p2], acc_sh.at[dsts[0]], sems[p2]).wait()

    MASK = jnp.int32(-65536)

    def _scale(j, p, p2):
        for g in range(CH // 16):
            ew16 = ew_v[pl.ds(j * CH + g * 16, 16)]
            for r in range(16):
                w = _lane_bcast(ew16, r)
                row = g * 16 + r
                for k in range(D // 32):
                    xi = bfr[p][row, pl.ds(k * 16, 16)]
                    lo = plsc.bitcast(xi << 16, jnp.float32)
                    hi = plsc.bitcast(xi & MASK, jnp.float32)
                    f32r[p2][row, pl.ds(k * 32, 16)] = lo * w
                    f32r[p2][row, pl.ds(k * 32 + 16, 16)] = hi * w

    def _stage(j, p, p2, do_sdrain=True, do_src4=True, do_g3=True,
               do_d2=True):
        q3 = (p + 3) % NBUF
        _wait_gather(p)
        if do_src4:
            _isrc(j + 4, p)
        if do_g3:
            _wait_isrc(q3)
            _gather(q3)
        if do_sdrain:
            _wait_scatter(p2)
        if do_d2:
            _idst(j + 2, (p + 2) % NBUF)
        _scale(j, p, p2)
        _wait_idst(p)
        _scatter(p, p2)

    for t in range(3):
        pltpu.sync_copy(src_hbm.at[pl.ds((row0 + t) * CH, CH)], srcs[t])
        _gather(t)
    _isrc(3, 3)
    _idst(0, 0)
    _idst(1, 1)
    _stage(0, 0, 0, do_sdrain=False)
    _stage(1, 1, 1, do_sdrain=False)

    def body(k, carry):
        j = 4 * k + 2
        for t in range(4):
            _stage(j + t, (2 + t) % NBUF, t % NSB)
        return carry

    lax.fori_loop(0, (nch - 8) // 4, body, 0)

    for i in range(6):
        _stage(nch - 6 + i, (i + 2) % NBUF, i % NSB, do_src4=(i < 2),
               do_g3=(i < 3), do_d2=(i < 4))
    _wait_scatter(0)
    _wait_scatter(1)
    plsc.subcore_barrier()

    for m in range(NPT // RB):
        stg = f32r[m % NSB]
        pltpu.sync_copy(acc_sh.at[pl.ds(s * NPT + m * RB, RB), :],
                        stg.at[pl.ds(0, RB), :])
        pltpu.sync_copy(stg.at[pl.ds(0, RB), :],
                        out_hbm.at[c, pl.ds(s * NPT + m * RB, RB), :])


def _interleave_bf16(hp):
    hpb = hp.reshape(N, D // 32, 2, 16).transpose(0, 1, 3, 2).reshape(
        N, D).astype(jnp.bfloat16)
    return lax.bitcast_convert_type(hpb.reshape(N, D // 2, 2),
                                    jnp.int32)


def _tc1_body(x_ref, w1_ref, degp_ref, hp1_ref, dinv_ref):
    deg = degp_ref[0] + degp_ref[1] + 1.0
    dinv = jnp.where(deg > 0, lax.rsqrt(deg), 0.0)
    h1 = jnp.dot(x_ref[...], w1_ref[...], preferred_element_type=jnp.float32)
    hp1_ref[...] = h1 * dinv
    dinv_ref[...] = dinv


def _tc2_body(accp_ref, hp_ref, dinv_ref, w_ref, b_ref, hpn_ref):
    dinv = dinv_ref[...]
    acc = accp_ref[0, :N, :] + accp_ref[1, :N, :]
    z = jnp.maximum(dinv * (acc + hp_ref[...]) + b_ref[...], 0.0)
    h = jnp.dot(z, w_ref[...], preferred_element_type=jnp.float32)
    hpn_ref[...] = h * dinv


def _tc3_body(accp_ref, hp_ref, dinv_ref, b2_ref, batch_ref,
              wf1_ref, bf1_ref, wf2_ref, bf2_ref, out_ref):
    dinv = dinv_ref[...]
    acc = accp_ref[0, :N, :] + accp_ref[1, :N, :]
    z = jnp.maximum(dinv * (acc + hp_ref[...]) + b2_ref[...], 0.0)
    gid = lax.broadcasted_iota(jnp.int32, (G, N), 0)
    ind = (gid == batch_ref[...]).astype(jnp.float32)
    sums = jnp.dot(ind, z, preferred_element_type=jnp.float32)
    counts = jnp.sum(ind, axis=1, keepdims=True)
    pooled = sums / jnp.maximum(counts, 1.0)
    g = jnp.maximum(jnp.dot(pooled, wf1_ref[...],
                            preferred_element_type=jnp.float32)
                    + bf1_ref[...], 0.0)
    out_ref[...] = jnp.dot(g, wf2_ref[...],
                           preferred_element_type=jnp.float32) + bf2_ref[...]


def _tc_call(body, out_shape, *args):
    return pl.pallas_call(
        body,
        out_shape=jax.ShapeDtypeStruct(out_shape, jnp.float32),
    )(*args)


@jax.jit
def kernel(x, edge_index, batch, edge_weight, W1, b1, W2, b2, Wf1, bf1, Wf2, bf2):
    pad = EP - E
    src = jnp.concatenate([edge_index[0], jnp.zeros((pad,), jnp.int32)])
    dst = jnp.concatenate([edge_index[1], jnp.zeros((pad,), jnp.int32)])
    ew = jnp.concatenate([edge_weight,
                          jnp.zeros((pad + NMX * CH,), jnp.float32)])

    degp = _sc_degree(dst.reshape(EP // DCH, DCH),
                      ew[:EP].reshape(EP // DCH, DCH))
    degp_col = degp.reshape(NC, NP, 1)[:, :N, :]

    hp1, dinv = pl.pallas_call(
        _tc1_body,
        out_shape=[
            jax.ShapeDtypeStruct((N, D), jnp.float32),
            jax.ShapeDtypeStruct((N, 1), jnp.float32),
        ],
    )(x, W1, degp_col)

    accp1 = _sc_messages(src, dst, ew, _interleave_bf16(hp1))
    hp2 = _tc_call(_tc2_body, (N, D), accp1, hp1, dinv, W2, b1.reshape(1, D))
    accp2 = _sc_messages(src, dst, ew, _interleave_bf16(hp2))
    out = _tc_call(_tc3_body, (G, D), accp2, hp2, dinv, b2.reshape(1, D),
                   batch.reshape(1, N), Wf1, bf1.reshape(1, D),
                   Wf2, bf2.reshape(1, D))
    return out


# speedup vs baseline: 1.1219x; 1.1219x over previous
import functools
import jax
import jax.numpy as jnp
from jax import lax
from jax.experimental import pallas as pl
from jax.experimental.pallas import tpu as pltpu
from jax.experimental.pallas import tpu_sc as plsc

N = 10000
E = 320000
D = 128
G = 16
NC = 2
NS = 16
NW = NC * NS
CH = 64
NCH = 160
DCH = 128
DNCH = 80
EP = NW * NCH * CH
NP = 10240
NPT = NP // NS
RB = 32

_mesh = plsc.VectorSubcoreMesh(core_axis_name="c", subcore_axis_name="s")

_GDN = lax.GatherDimensionNumbers(
    offset_dims=(), collapsed_slice_dims=(0,), start_index_map=(0,))


def _lane_bcast(vec16, r):
    idx = jnp.full((16, 1), r, jnp.int32)
    return lax.gather(vec16, idx, _GDN, (1,),
                      mode=lax.GatherScatterMode.PROMISE_IN_BOUNDS)


@functools.partial(
    pl.kernel,
    out_type=jax.ShapeDtypeStruct((NC, NP), jnp.float32),
    mesh=_mesh,
    scratch_types=[
        pltpu.VMEM((DNCH, DCH), jnp.int32),
        pltpu.VMEM((DNCH, DCH), jnp.float32),
        pltpu.VMEM((NPT,), jnp.float32),
        pltpu.VMEM_SHARED((NP,), jnp.float32),
    ],
)
def _sc_degree(dst_hbm, ew_hbm, out_hbm, dst_v, ew_v, buf_v, deg_sh):
    c = lax.axis_index("c")
    s = lax.axis_index("s")
    wid = c * NS + s

    for k in range(NPT // 16):
        buf_v[pl.ds(k * 16, 16)] = jnp.zeros((16,), jnp.float32)
    pltpu.sync_copy(buf_v, deg_sh.at[pl.ds(s * NPT, NPT)])
    plsc.subcore_barrier()

    row0 = wid * DNCH
    pltpu.sync_copy(dst_hbm.at[pl.ds(row0, DNCH), :], dst_v)
    pltpu.sync_copy(ew_hbm.at[pl.ds(row0, DNCH), :], ew_v)

    def body(j, carry):
        pltpu.sync_copy(ew_v.at[j], deg_sh.at[dst_v.at[j]], add=True)
        return carry

    lax.fori_loop(0, DNCH, body, 0)
    plsc.subcore_barrier()

    pltpu.sync_copy(deg_sh.at[pl.ds(s * NPT, NPT)], buf_v)
    pltpu.sync_copy(buf_v, out_hbm.at[c, pl.ds(s * NPT, NPT)])


NBUF = 4
NSB = 2
N0 = 204
N1 = 116
NMX = max(N0, N1)

_msg_scratch = (
    [pltpu.VMEM((NMX * CH,), jnp.float32)]
    + [pltpu.VMEM((CH,), jnp.int32) for _ in range(NBUF)]
    + [pltpu.VMEM((CH,), jnp.int32) for _ in range(NBUF)]
    + [pltpu.VMEM((CH, D // 2), jnp.int32) for _ in range(NBUF)]
    + [pltpu.VMEM((CH, D), jnp.float32) for _ in range(NSB)]
    + [pltpu.VMEM_SHARED((NP, D), jnp.float32)]
    + [pltpu.SemaphoreType.DMA for _ in range(3 * NBUF + NSB)]
)


@functools.partial(
    pl.kernel,
    out_type=jax.ShapeDtypeStruct((NC, NP, D), jnp.float32),
    mesh=_mesh,
    scratch_types=_msg_scratch,
    compiler_params=pltpu.CompilerParams(needs_layout_passes=False, use_tc_tiling_on_sc=False),
)
def _sc_messages(src_hbm, dst_hbm, ew_hbm, hpb_hbm, out_hbm, ew_v, *bufs):
    srcs = bufs[0:NBUF]
    dsts = bufs[NBUF:2 * NBUF]
    bfr = bufs[2 * NBUF:3 * NBUF]
    f32r = bufs[3 * NBUF:3 * NBUF + NSB]
    acc_sh = bufs[3 * NBUF + NSB]
    o = 3 * NBUF + NSB + 1
    semg = bufs[o:o + NBUF]
    semis = bufs[o + NBUF:o + 2 * NBUF]
    semid = bufs[o + 2 * NBUF:o + 3 * NBUF]
    sems = bufs[o + 3 * NBUF:o + 3 * NBUF + NSB]

    c = lax.axis_index("c")
    s = lax.axis_index("s")
    nch = jnp.where(c == 0, N0, N1)
    row0 = jnp.where(c == 0, s * N0, NS * N0 + s * N1)

    for r in range(RB):
        for k in range(D // 16):
            f32r[0][r, pl.ds(k * 16, 16)] = jnp.zeros((16,), jnp.float32)
    for m in range(NPT // RB):
        pltpu.sync_copy(f32r[0].at[pl.ds(0, RB), :],
                        acc_sh.at[pl.ds(s * NPT + m * RB, RB), :])
    plsc.subcore_barrier()

    ewoff = NMX - nch
    pltpu.sync_copy(ew_hbm.at[pl.ds((row0 - ewoff) * CH, NMX * CH)], ew_v)

    def _isrc(j, p):
        pltpu.async_copy(src_hbm.at[pl.ds((row0 + j) * CH, CH)],
                         srcs[p], semis[p])

    def _idst(j, p):
        pltpu.async_copy(dst_hbm.at[pl.ds((row0 + j) * CH, CH)],
                         dsts[p], semid[p])

    def _gather(p):
        pltpu.async_copy(hpb_hbm.at[srcs[p]], bfr[p], semg[p])

    def _scatter(p, p2):
        pltpu.async_copy(f32r[p2], acc_sh.at[dsts[p]], sems[p2], add=True)

    def _wait_isrc(p):
        pltpu.make_async_copy(src_hbm.at[pl.ds(0, CH)], srcs[p],
                              semis[p]).wait()

    def _wait_idst(p):
        pltpu.make_async_copy(dst_hbm.at[pl.ds(0, CH)], dsts[p],
                              semid[p]).wait()

    def _wait_gather(p):
        pltpu.make_async_copy(hpb_hbm.at[srcs[p]], bfr[p], semg[p]).wait()

    def _wait_scatter(p2):
        pltpu.make_async_copy(f32r[p2], acc_sh.at[dsts[0]], sems[p2]).wait()

    MASK = jnp.int32(-65536)

    def _scale(j, p, p2):
        for g in range(CH // 16):
            ew16 = ew_v[pl.ds((j + ewoff) * CH + g * 16, 16)]
            for r in range(16):
                w = _lane_bcast(ew16, r)
                row = g * 16 + r
                for k in range(D // 32):
                    xi = bfr[p][row, pl.ds(k * 16, 16)]
                    lo = plsc.bitcast(xi << 16, jnp.float32)
                    hi = plsc.bitcast(xi & MASK, jnp.float32)
                    f32r[p2][row, pl.ds(k * 32, 16)] = lo * w
                    f32r[p2][row, pl.ds(k * 32 + 16, 16)] = hi * w

    def _stage(j, p, p2, do_sdrain=True, do_src4=True, do_g3=True,
               do_d2=True):
        q3 = (p + 3) % NBUF
        _wait_gather(p)
        if do_src4:
            _isrc(j + 4, p)
        if do_g3:
            _wait_isrc(q3)
            _gather(q3)
        if do_sdrain:
            _wait_scatter(p2)
        if do_d2:
            _idst(j + 2, (p + 2) % NBUF)
        _scale(j, p, p2)
        _wait_idst(p)
        _scatter(p, p2)

    for t in range(3):
        pltpu.sync_copy(src_hbm.at[pl.ds((row0 + t) * CH, CH)], srcs[t])
        _gather(t)
    _isrc(3, 3)
    _idst(0, 0)
    _idst(1, 1)
    _stage(0, 0, 0, do_sdrain=False)
    _stage(1, 1, 1, do_sdrain=False)

    def body(k, carry):
        j = 4 * k + 2
        for t in range(4):
            _stage(j + t, (2 + t) % NBUF, t % NSB)
        return carry

    lax.fori_loop(0, (nch - 8) // 4, body, 0)

    for i in range(6):
        _stage(nch - 6 + i, (i + 2) % NBUF, i % NSB, do_src4=(i < 2),
               do_g3=(i < 3), do_d2=(i < 4))
    _wait_scatter(0)
    _wait_scatter(1)
    plsc.subcore_barrier()

    for m in range(NPT // RB):
        stg = f32r[m % NSB]
        pltpu.sync_copy(acc_sh.at[pl.ds(s * NPT + m * RB, RB), :],
                        stg.at[pl.ds(0, RB), :])
        pltpu.sync_copy(stg.at[pl.ds(0, RB), :],
                        out_hbm.at[c, pl.ds(s * NPT + m * RB, RB), :])


def _interleave_bf16(hp):
    hpb = hp.reshape(N, D // 32, 2, 16).transpose(0, 1, 3, 2).reshape(
        N, D).astype(jnp.bfloat16)
    return lax.bitcast_convert_type(hpb.reshape(N, D // 2, 2),
                                    jnp.int32)


def _tc1_body(x_ref, w1_ref, degp_ref, hp1_ref, dinv_ref):
    deg = degp_ref[0] + degp_ref[1] + 1.0
    dinv = jnp.where(deg > 0, lax.rsqrt(deg), 0.0)
    h1 = jnp.dot(x_ref[...], w1_ref[...], preferred_element_type=jnp.float32)
    hp1_ref[...] = h1 * dinv
    dinv_ref[...] = dinv


def _tc2_body(accp_ref, hp_ref, dinv_ref, w_ref, b_ref, hpn_ref):
    dinv = dinv_ref[...]
    acc = accp_ref[0, :N, :] + accp_ref[1, :N, :]
    z = jnp.maximum(dinv * (acc + hp_ref[...]) + b_ref[...], 0.0)
    h = jnp.dot(z, w_ref[...], preferred_element_type=jnp.float32)
    hpn_ref[...] = h * dinv


def _tc3_body(accp_ref, hp_ref, dinv_ref, b2_ref, batch_ref,
              wf1_ref, bf1_ref, wf2_ref, bf2_ref, out_ref):
    dinv = dinv_ref[...]
    acc = accp_ref[0, :N, :] + accp_ref[1, :N, :]
    z = jnp.maximum(dinv * (acc + hp_ref[...]) + b2_ref[...], 0.0)
    gid = lax.broadcasted_iota(jnp.int32, (G, N), 0)
    ind = (gid == batch_ref[...]).astype(jnp.float32)
    sums = jnp.dot(ind, z, preferred_element_type=jnp.float32)
    counts = jnp.sum(ind, axis=1, keepdims=True)
    pooled = sums / jnp.maximum(counts, 1.0)
    g = jnp.maximum(jnp.dot(pooled, wf1_ref[...],
                            preferred_element_type=jnp.float32)
                    + bf1_ref[...], 0.0)
    out_ref[...] = jnp.dot(g, wf2_ref[...],
                           preferred_element_type=jnp.float32) + bf2_ref[...]


def _tc_call(body, out_shape, *args):
    return pl.pallas_call(
        body,
        out_shape=jax.ShapeDtypeStruct(out_shape, jnp.float32),
    )(*args)


@jax.jit
def kernel(x, edge_index, batch, edge_weight, W1, b1, W2, b2, Wf1, bf1, Wf2, bf2):
    pad = EP - E
    src = jnp.concatenate([edge_index[0], jnp.zeros((pad,), jnp.int32)])
    dst = jnp.concatenate([edge_index[1], jnp.zeros((pad,), jnp.int32)])
    ew = jnp.concatenate([edge_weight, jnp.zeros((pad,), jnp.float32)])

    degp = _sc_degree(dst.reshape(EP // DCH, DCH),
                      ew.reshape(EP // DCH, DCH))
    degp_col = degp.reshape(NC, NP, 1)[:, :N, :]

    hp1, dinv = pl.pallas_call(
        _tc1_body,
        out_shape=[
            jax.ShapeDtypeStruct((N, D), jnp.float32),
            jax.ShapeDtypeStruct((N, 1), jnp.float32),
        ],
    )(x, W1, degp_col)

    accp1 = _sc_messages(src, dst, ew, _interleave_bf16(hp1))
    hp2 = _tc_call(_tc2_body, (N, D), accp1, hp1, dinv, W2, b1.reshape(1, D))
    accp2 = _sc_messages(src, dst, ew, _interleave_bf16(hp2))
    out = _tc_call(_tc3_body, (G, D), accp2, hp2, dinv, b2.reshape(1, D),
                   batch.reshape(1, N), Wf1, bf1.reshape(1, D),
                   Wf2, bf2.reshape(1, D))
    return out


# speedup vs baseline: 1.1276x; 1.0050x over previous
import functools
import jax
import jax.numpy as jnp
from jax import lax
from jax.experimental import pallas as pl
from jax.experimental.pallas import tpu as pltpu
from jax.experimental.pallas import tpu_sc as plsc

N = 10000
E = 320000
D = 128
G = 16
NC = 2
NS = 16
NW = NC * NS
CH = 64
NCH = 160
DCH = 128
DNCH = 80
EP = NW * NCH * CH
NP = 10240
NPT = NP // NS
RB = 32

_mesh = plsc.VectorSubcoreMesh(core_axis_name="c", subcore_axis_name="s")

_GDN = lax.GatherDimensionNumbers(
    offset_dims=(), collapsed_slice_dims=(0,), start_index_map=(0,))


def _lane_bcast(vec16, r):
    idx = jnp.full((16, 1), r, jnp.int32)
    return lax.gather(vec16, idx, _GDN, (1,),
                      mode=lax.GatherScatterMode.PROMISE_IN_BOUNDS)


@functools.partial(
    pl.kernel,
    out_type=jax.ShapeDtypeStruct((NC, NP), jnp.float32),
    mesh=_mesh,
    scratch_types=[
        pltpu.VMEM((DNCH, DCH), jnp.int32),
        pltpu.VMEM((DNCH, DCH), jnp.float32),
        pltpu.VMEM((NPT,), jnp.float32),
        pltpu.VMEM_SHARED((NP,), jnp.float32),
    ],
)
def _sc_degree(dst_hbm, ew_hbm, out_hbm, dst_v, ew_v, buf_v, deg_sh):
    c = lax.axis_index("c")
    s = lax.axis_index("s")
    wid = c * NS + s

    for k in range(NPT // 16):
        buf_v[pl.ds(k * 16, 16)] = jnp.zeros((16,), jnp.float32)
    pltpu.sync_copy(buf_v, deg_sh.at[pl.ds(s * NPT, NPT)])
    plsc.subcore_barrier()

    row0 = wid * DNCH
    pltpu.sync_copy(dst_hbm.at[pl.ds(row0, DNCH), :], dst_v)
    pltpu.sync_copy(ew_hbm.at[pl.ds(row0, DNCH), :], ew_v)

    def body(j, carry):
        pltpu.sync_copy(ew_v.at[j], deg_sh.at[dst_v.at[j]], add=True)
        return carry

    lax.fori_loop(0, DNCH, body, 0)
    plsc.subcore_barrier()

    pltpu.sync_copy(deg_sh.at[pl.ds(s * NPT, NPT)], buf_v)
    pltpu.sync_copy(buf_v, out_hbm.at[c, pl.ds(s * NPT, NPT)])


NBUF = 4
NSB = 2
N0 = 212
N1 = 108
NMX = max(N0, N1)

_msg_scratch = (
    [pltpu.VMEM((NMX * CH,), jnp.float32)]
    + [pltpu.VMEM((CH,), jnp.int32) for _ in range(NBUF)]
    + [pltpu.VMEM((CH,), jnp.int32) for _ in range(NBUF)]
    + [pltpu.VMEM((CH, D // 2), jnp.int32) for _ in range(NBUF)]
    + [pltpu.VMEM((CH, D), jnp.float32) for _ in range(NSB)]
    + [pltpu.VMEM_SHARED((NP, D), jnp.float32)]
    + [pltpu.SemaphoreType.DMA for _ in range(3 * NBUF + NSB)]
)


@functools.partial(
    pl.kernel,
    out_type=jax.ShapeDtypeStruct((NC, NP, D), jnp.float32),
    mesh=_mesh,
    scratch_types=_msg_scratch,
    compiler_params=pltpu.CompilerParams(needs_layout_passes=False, use_tc_tiling_on_sc=False),
)
def _sc_messages(src_hbm, dst_hbm, ew_hbm, hpb_hbm, out_hbm, ew_v, *bufs):
    srcs = bufs[0:NBUF]
    dsts = bufs[NBUF:2 * NBUF]
    bfr = bufs[2 * NBUF:3 * NBUF]
    f32r = bufs[3 * NBUF:3 * NBUF + NSB]
    acc_sh = bufs[3 * NBUF + NSB]
    o = 3 * NBUF + NSB + 1
    semg = bufs[o:o + NBUF]
    semis = bufs[o + NBUF:o + 2 * NBUF]
    semid = bufs[o + 2 * NBUF:o + 3 * NBUF]
    sems = bufs[o + 3 * NBUF:o + 3 * NBUF + NSB]

    c = lax.axis_index("c")
    s = lax.axis_index("s")
    nch = jnp.where(c == 0, N0, N1)
    row0 = jnp.where(c == 0, s * N0, NS * N0 + s * N1)

    for r in range(RB):
        for k in range(D // 16):
            f32r[0][r, pl.ds(k * 16, 16)] = jnp.zeros((16,), jnp.float32)
    for m in range(NPT // RB):
        pltpu.sync_copy(f32r[0].at[pl.ds(0, RB), :],
                        acc_sh.at[pl.ds(s * NPT + m * RB, RB), :])
    plsc.subcore_barrier()

    ewoff = NMX - nch
    pltpu.sync_copy(ew_hbm.at[pl.ds((row0 - ewoff) * CH, NMX * CH)], ew_v)

    def _isrc(j, p):
        pltpu.async_copy(src_hbm.at[pl.ds((row0 + j) * CH, CH)],
                         srcs[p], semis[p])

    def _idst(j, p):
        pltpu.async_copy(dst_hbm.at[pl.ds((row0 + j) * CH, CH)],
                         dsts[p], semid[p])

    def _gather(p):
        pltpu.async_copy(hpb_hbm.at[srcs[p]], bfr[p], semg[p])

    def _scatter(p, p2):
        pltpu.async_copy(f32r[p2], acc_sh.at[dsts[p]], sems[p2], add=True)

    def _wait_isrc(p):
        pltpu.make_async_copy(src_hbm.at[pl.ds(0, CH)], srcs[p],
                              semis[p]).wait()

    def _wait_idst(p):
        pltpu.make_async_copy(dst_hbm.at[pl.ds(0, CH)], dsts[p],
                              semid[p]).wait()

    def _wait_gather(p):
        pltpu.make_async_copy(hpb_hbm.at[srcs[p]], bfr[p], semg[p]).wait()

    def _wait_scatter(p2):
        pltpu.make_async_copy(f32r[p2], acc_sh.at[dsts[0]], sems[p2]).wait()

    MASK = jnp.int32(-65536)

    def _scale(j, p, p2):
        for g in range(CH // 16):
            ew16 = ew_v[pl.ds((j + ewoff) * CH + g * 16, 16)]
            for r in range(16):
                w = _lane_bcast(ew16, r)
                row = g * 16 + r
                for k in range(D // 32):
                    xi = bfr[p][row, pl.ds(k * 16, 16)]
                    lo = plsc.bitcast(xi << 16, jnp.float32)
                    hi = plsc.bitcast(xi & MASK, jnp.float32)
                    f32r[p2][row, pl.ds(k * 32, 16)] = lo * w
                    f32r[p2][row, pl.ds(k * 32 + 16, 16)] = hi * w

    def _stage(j, p, p2, do_sdrain=True, do_src4=True, do_g3=True,
               do_d2=True):
        q3 = (p + 3) % NBUF
        _wait_gather(p)
        if do_src4:
            _isrc(j + 4, p)
        if do_g3:
            _wait_isrc(q3)
            _gather(q3)
        if do_sdrain:
            _wait_scatter(p2)
        if do_d2:
            _idst(j + 2, (p + 2) % NBUF)
        _scale(j, p, p2)
        _wait_idst(p)
        _scatter(p, p2)

    for t in range(3):
        pltpu.sync_copy(src_hbm.at[pl.ds((row0 + t) * CH, CH)], srcs[t])
        _gather(t)
    _isrc(3, 3)
    _idst(0, 0)
    _idst(1, 1)
    _stage(0, 0, 0, do_sdrain=False)
    _stage(1, 1, 1, do_sdrain=False)

    def body(k, carry):
        j = 4 * k + 2
        for t in range(4):
            _stage(j + t, (2 + t) % NBUF, t % NSB)
        return carry

    lax.fori_loop(0, (nch - 8) // 4, body, 0)

    for i in range(6):
        _stage(nch - 6 + i, (i + 2) % NBUF, i % NSB, do_src4=(i < 2),
               do_g3=(i < 3), do_d2=(i < 4))
    _wait_scatter(0)
    _wait_scatter(1)
    plsc.subcore_barrier()

    for m in range(NPT // RB):
        stg = f32r[m % NSB]
        pltpu.sync_copy(acc_sh.at[pl.ds(s * NPT + m * RB, RB), :],
                        stg.at[pl.ds(0, RB), :])
        pltpu.sync_copy(stg.at[pl.ds(0, RB), :],
                        out_hbm.at[c, pl.ds(s * NPT + m * RB, RB), :])


def _interleave_bf16(hp):
    hpb = hp.reshape(N, D // 32, 2, 16).transpose(0, 1, 3, 2).reshape(
        N, D).astype(jnp.bfloat16)
    return lax.bitcast_convert_type(hpb.reshape(N, D // 2, 2),
                                    jnp.int32)


def _tc1_body(x_ref, w1_ref, degp_ref, hp1_ref, dinv_ref):
    deg = degp_ref[0] + degp_ref[1] + 1.0
    dinv = jnp.where(deg > 0, lax.rsqrt(deg), 0.0)
    h1 = jnp.dot(x_ref[...], w1_ref[...], preferred_element_type=jnp.float32)
    hp1_ref[...] = h1 * dinv
    dinv_ref[...] = dinv


def _tc2_body(accp_ref, hp_ref, dinv_ref, w_ref, b_ref, hpn_ref):
    dinv = dinv_ref[...]
    acc = accp_ref[0, :N, :] + accp_ref[1, :N, :]
    z = jnp.maximum(dinv * (acc + hp_ref[...]) + b_ref[...], 0.0)
    h = jnp.dot(z, w_ref[...], preferred_element_type=jnp.float32)
    hpn_ref[...] = h * dinv


def _tc3_body(accp_ref, hp_ref, dinv_ref, b2_ref, batch_ref,
              wf1_ref, bf1_ref, wf2_ref, bf2_ref, out_ref):
    dinv = dinv_ref[...]
    acc = accp_ref[0, :N, :] + accp_ref[1, :N, :]
    z = jnp.maximum(dinv * (acc + hp_ref[...]) + b2_ref[...], 0.0)
    gid = lax.broadcasted_iota(jnp.int32, (G, N), 0)
    ind = (gid == batch_ref[...]).astype(jnp.float32)
    sums = jnp.dot(ind, z, preferred_element_type=jnp.float32)
    counts = jnp.sum(ind, axis=1, keepdims=True)
    pooled = sums / jnp.maximum(counts, 1.0)
    g = jnp.maximum(jnp.dot(pooled, wf1_ref[...],
                            preferred_element_type=jnp.float32)
                    + bf1_ref[...], 0.0)
    out_ref[...] = jnp.dot(g, wf2_ref[...],
                           preferred_element_type=jnp.float32) + bf2_ref[...]


def _tc_call(body, out_shape, *args):
    return pl.pallas_call(
        body,
        out_shape=jax.ShapeDtypeStruct(out_shape, jnp.float32),
    )(*args)


@jax.jit
def kernel(x, edge_index, batch, edge_weight, W1, b1, W2, b2, Wf1, bf1, Wf2, bf2):
    pad = EP - E
    src = jnp.concatenate([edge_index[0], jnp.zeros((pad,), jnp.int32)])
    dst = jnp.concatenate([edge_index[1], jnp.zeros((pad,), jnp.int32)])
    ew = jnp.concatenate([edge_weight, jnp.zeros((pad,), jnp.float32)])

    degp = _sc_degree(dst.reshape(EP // DCH, DCH),
                      ew.reshape(EP // DCH, DCH))
    degp_col = degp.reshape(NC, NP, 1)[:, :N, :]

    hp1, dinv = pl.pallas_call(
        _tc1_body,
        out_shape=[
            jax.ShapeDtypeStruct((N, D), jnp.float32),
            jax.ShapeDtypeStruct((N, 1), jnp.float32),
        ],
    )(x, W1, degp_col)

    accp1 = _sc_messages(src, dst, ew, _interleave_bf16(hp1))
    hp2 = _tc_call(_tc2_body, (N, D), accp1, hp1, dinv, W2, b1.reshape(1, D))
    accp2 = _sc_messages(src, dst, ew, _interleave_bf16(hp2))
    out = _tc_call(_tc3_body, (G, D), accp2, hp2, dinv, b2.reshape(1, D),
                   batch.reshape(1, N), Wf1, bf1.reshape(1, D),
                   Wf2, bf2.reshape(1, D))
    return out
